# Initial kernel scaffold; baseline (speedup 1.0000x reference)
import functools

import jax
import jax.numpy as jnp
from jax import lax
from jax.experimental import pallas as pl
from jax.experimental.pallas import tpu as pltpu
from jax.experimental.pallas import tpu_sc as plsc

N = 10000
E = 320000
NSUB = 16
NCORE = 2
B = 128
TPB = 158
EP = NSUB * TPB * B
NPAD = 10016
RPT = NPAD // NSUB


def _make_mp(f_sc):
    mesh = plsc.VectorSubcoreMesh(core_axis_name="c", subcore_axis_name="s")
    fs = jax.ShapeDtypeStruct((NPAD, f_sc), jnp.float32)

    @functools.partial(
        pl.kernel,
        out_type=(fs, fs),
        mesh=mesh,
        scratch_types=[
            pltpu.VMEM((TPB, B), jnp.int32),
            pltpu.VMEM((TPB, B), jnp.int32),
            pltpu.VMEM((B, f_sc), jnp.float32),
            pltpu.VMEM((B, f_sc), jnp.float32),
            pltpu.VMEM_SHARED((NPAD, f_sc), jnp.float32),
            pltpu.SemaphoreType.DMA,
            pltpu.SemaphoreType.DMA,
        ],
    )
    def mp(h_a, h_b, src2d, dst2d, zrows, out_a, out_b,
           sidx, didx, rows0, rows1, acc, sem0, sem1):
        c = lax.axis_index("c")
        s = lax.axis_index("s")
        r0 = s * RPT
        pltpu.sync_copy(zrows.at[pl.ds(r0, RPT)], acc.at[pl.ds(r0, RPT)])
        pltpu.sync_copy(src2d.at[pl.ds(s * TPB, TPB)], sidx)
        pltpu.sync_copy(dst2d.at[pl.ds(s * TPB, TPB)], didx)
        plsc.subcore_barrier()

        for ci, h in ((0, h_a), (1, h_b)):
            @pl.when(c == ci)
            def _():
                pltpu.async_copy(h.at[sidx.at[0]], rows0, sem0)
                pltpu.async_copy(h.at[sidx.at[1]], rows1, sem1)

                def body(k, carry):
                    j0 = 2 * k
                    pltpu.make_async_copy(h.at[sidx.at[j0]], rows0, sem0).wait()
                    pltpu.sync_copy(rows0, acc.at[didx.at[j0]], add=True)
                    pltpu.async_copy(h.at[sidx.at[j0 + 2]], rows0, sem0)
                    pltpu.make_async_copy(h.at[sidx.at[j0 + 1]], rows1, sem1).wait()
                    pltpu.sync_copy(rows1, acc.at[didx.at[j0 + 1]], add=True)
                    pltpu.async_copy(h.at[sidx.at[j0 + 3]], rows1, sem1)
                    return carry

                lax.fori_loop(0, TPB // 2 - 1, body, 0)
                pltpu.make_async_copy(h.at[sidx.at[TPB - 2]], rows0, sem0).wait()
                pltpu.sync_copy(rows0, acc.at[didx.at[TPB - 2]], add=True)
                pltpu.make_async_copy(h.at[sidx.at[TPB - 1]], rows1, sem1).wait()
                pltpu.sync_copy(rows1, acc.at[didx.at[TPB - 1]], add=True)

        plsc.subcore_barrier()
        for ci, out in ((0, out_a), (1, out_b)):
            @pl.when(c == ci)
            def _():
                pltpu.sync_copy(acc.at[pl.ds(r0, RPT)], out.at[pl.ds(r0, RPT)])

    return mp


BR = 1000


def _stage_a(s1a, s1b, w1a, w1b, w2bd):
    def body(s1a_ref, s1b_ref, w1a_ref, w1b_ref, w2_ref, ga_ref, gb_ref):
        t = jnp.tanh(
            jnp.dot(s1a_ref[...], w1a_ref[...], preferred_element_type=jnp.float32)
            + jnp.dot(s1b_ref[...], w1b_ref[...], preferred_element_type=jnp.float32)
        )
        g = jnp.dot(t, w2_ref[...], preferred_element_type=jnp.float32)
        ga_ref[...] = g[:, :128]
        gb_ref[...] = g[:, 128:]

    return pl.pallas_call(
        body,
        grid=(N // BR,),
        in_specs=[
            pl.BlockSpec((BR, 64), lambda i: (i, 0)),
            pl.BlockSpec((BR, 64), lambda i: (i, 0)),
            pl.BlockSpec((64, 256), lambda i: (0, 0)),
            pl.BlockSpec((64, 256), lambda i: (0, 0)),
            pl.BlockSpec((256, 256), lambda i: (0, 0)),
        ],
        out_specs=[
            pl.BlockSpec((BR, 128), lambda i: (i, 0)),
            pl.BlockSpec((BR, 128), lambda i: (i, 0)),
        ],
        out_shape=[
            jax.ShapeDtypeStruct((N, 128), jnp.float32),
            jax.ShapeDtypeStruct((N, 128), jnp.float32),
        ],
    )(s1a, s1b, w1a, w1b, w2bd)


def _stage_b(s2a, s2b, wca, wcb):
    def body(s2a_ref, s2b_ref, wca_ref, wcb_ref, pa_ref, pb_ref):
        p = jnp.dot(jnp.tanh(s2a_ref[...]), wca_ref[...],
                    preferred_element_type=jnp.float32)
        p += jnp.dot(jnp.tanh(s2b_ref[...]), wcb_ref[...],
                     preferred_element_type=jnp.float32)
        pa_ref[...] = p[:, :32]
        pb_ref[...] = p[:, 32:]

    return pl.pallas_call(
        body,
        grid=(N // BR,),
        in_specs=[
            pl.BlockSpec((BR, 128), lambda i: (i, 0)),
            pl.BlockSpec((BR, 128), lambda i: (i, 0)),
            pl.BlockSpec((128, 64), lambda i: (0, 0)),
            pl.BlockSpec((128, 64), lambda i: (0, 0)),
        ],
        out_specs=[
            pl.BlockSpec((BR, 32), lambda i: (i, 0)),
            pl.BlockSpec((BR, 32), lambda i: (i, 0)),
        ],
        out_shape=[
            jax.ShapeDtypeStruct((N, 32), jnp.float32),
            jax.ShapeDtypeStruct((N, 32), jnp.float32),
        ],
    )(s2a, s2b, wca, wcb)


def _stage_c(s3a, s3b, eps):
    def body(s3a_ref, s3b_ref, eps_ref, z_ref, m_ref, std_ref):
        m = s3a_ref[...]
        std = jnp.maximum(s3b_ref[...], 0.0) + 0.0001
        z_ref[...] = eps_ref[...] * std + m
        m_ref[...] = m
        std_ref[...] = std

    return pl.pallas_call(
        body,
        grid=(N // BR,),
        in_specs=[
            pl.BlockSpec((BR, 32), lambda i: (i, 0)),
            pl.BlockSpec((BR, 32), lambda i: (i, 0)),
            pl.BlockSpec((BR, 32), lambda i: (i, 0)),
        ],
        out_specs=[
            pl.BlockSpec((BR, 32), lambda i: (i, 0)),
            pl.BlockSpec((BR, 32), lambda i: (i, 0)),
            pl.BlockSpec((BR, 32), lambda i: (i, 0)),
        ],
        out_shape=[
            jax.ShapeDtypeStruct((N, 32), jnp.float32),
            jax.ShapeDtypeStruct((N, 32), jnp.float32),
            jax.ShapeDtypeStruct((N, 32), jnp.float32),
        ],
    )(s3a, s3b, eps)


def kernel(x, edge_index, W1_0, W1_1, W1_2, W1_3, W2_0, W2_1, W2_2, W2_3,
           Wm, Ws, eps):
    src = edge_index[0]
    dst = edge_index[1]
    pad = EP - E
    src2d = jnp.concatenate(
        [src, jnp.zeros((pad,), jnp.int32)]).reshape(EP // B, B)
    dst2d = jnp.concatenate(
        [dst, jnp.full((pad,), N, jnp.int32)]).reshape(EP // B, B)

    w1cat = jnp.concatenate([W1_0, W1_1, W1_2, W1_3], axis=1)
    z64 = jnp.zeros((64, 64), jnp.float32)
    w2bd = jnp.block([
        [W2_0, z64, z64, z64],
        [z64, W2_1, z64, z64],
        [z64, z64, W2_2, z64],
        [z64, z64, z64, W2_3],
    ])
    wcat = jnp.concatenate([Wm, Ws], axis=1)

    mp64 = _make_mp(64)
    mp128 = _make_mp(128)
    mp32 = _make_mp(32)

    s1a, s1b = mp64(x[:, :64], x[:, 64:], src2d, dst2d,
                    jnp.zeros((NPAD, 64), jnp.float32))
    ga, gb = _stage_a(s1a[:N], s1b[:N], w1cat[:64], w1cat[64:], w2bd)

    s2a, s2b = mp128(ga, gb, src2d, dst2d,
                     jnp.zeros((NPAD, 128), jnp.float32))
    pa, pb = _stage_b(s2a[:N], s2b[:N], wcat[:128], wcat[128:])

    s3a, s3b = mp32(pa, pb, src2d, dst2d,
                    jnp.zeros((NPAD, 32), jnp.float32))
    z, m_q_z, std_q_z = _stage_c(s3a[:N], s3b[:N], eps)
    return (z, m_q_z, std_q_z)


# speedup vs baseline: 3.8192x; 3.8192x over previous
import functools

import jax
import jax.numpy as jnp
from jax import lax
from jax.experimental import pallas as pl
from jax.experimental.pallas import tpu as pltpu
from jax.experimental.pallas import tpu_sc as plsc

N = 10000
E = 320000
NSUB = 16
NCORE = 2
B = 128
NB = 2560
EP = NB * B
TPB = NB // NSUB
F = 128

NT = 10240
HALF = NT // 2
HR = 5248
DUMMY = HALF
RPTA = HR // NSUB
RPTO = HALF // NSUB


def _make_mp(table_rows):
    mesh = plsc.VectorSubcoreMesh(core_axis_name="c", subcore_axis_name="s")

    @functools.partial(
        pl.kernel,
        out_type=jax.ShapeDtypeStruct((NT, F), jnp.float32),
        mesh=mesh,
        scratch_types=[
            pltpu.VMEM((TPB, B), jnp.int32),
            pltpu.VMEM((TPB, B), jnp.int32),
            pltpu.VMEM((B, F), jnp.float32),
            pltpu.VMEM((B, F), jnp.float32),
            pltpu.VMEM_SHARED((HR, F), jnp.float32),
            pltpu.SemaphoreType.DMA,
            pltpu.SemaphoreType.DMA,
        ],
    )
    def mp(h, dsta2d, dstb2d, src2d, zrows, out,
           sidx, didx, rows0, rows1, acc, sem0, sem1):
        c = lax.axis_index("c")
        s = lax.axis_index("s")
        base = s * TPB
        pltpu.sync_copy(zrows.at[pl.ds(s * RPTA, RPTA)],
                        acc.at[pl.ds(s * RPTA, RPTA)])
        pltpu.sync_copy(src2d.at[pl.ds(base, TPB)], sidx)
        for ci, d2d in ((0, dsta2d), (1, dstb2d)):
            @pl.when(c == ci)
            def _():
                pltpu.sync_copy(d2d.at[pl.ds(base, TPB)], didx)
        plsc.subcore_barrier()

        pltpu.async_copy(h.at[sidx.at[0]], rows0, sem0)
        pltpu.async_copy(h.at[sidx.at[1]], rows1, sem1)

        def body(k, carry):
            j0 = 2 * k
            pltpu.make_async_copy(h.at[sidx.at[j0]], rows0, sem0).wait()
            pltpu.sync_copy(rows0, acc.at[didx.at[j0]], add=True)
            pltpu.async_copy(h.at[sidx.at[j0 + 2]], rows0, sem0)
            pltpu.make_async_copy(h.at[sidx.at[j0 + 1]], rows1, sem1).wait()
            pltpu.sync_copy(rows1, acc.at[didx.at[j0 + 1]], add=True)
            pltpu.async_copy(h.at[sidx.at[j0 + 3]], rows1, sem1)
            return carry

        lax.fori_loop(0, TPB // 2 - 1, body, 0)
        pltpu.make_async_copy(h.at[sidx.at[TPB - 2]], rows0, sem0).wait()
        pltpu.sync_copy(rows0, acc.at[didx.at[TPB - 2]], add=True)
        pltpu.make_async_copy(h.at[sidx.at[TPB - 1]], rows1, sem1).wait()
        pltpu.sync_copy(rows1, acc.at[didx.at[TPB - 1]], add=True)

        plsc.subcore_barrier()
        pltpu.sync_copy(acc.at[pl.ds(s * RPTO, RPTO)],
                        out.at[pl.ds(c * HALF + s * RPTO, RPTO)])

    return mp


BR = 1024


def _stage_a(s1, w1cat, w2bd):
    def body(s1_ref, w1_ref, w2_ref, ga_ref, gb_ref):
        t = jnp.tanh(jnp.dot(s1_ref[...], w1_ref[...],
                             preferred_element_type=jnp.float32))
        g = jnp.dot(t, w2_ref[...], preferred_element_type=jnp.float32)
        ga_ref[...] = g[:, :128]
        gb_ref[...] = g[:, 128:]

    out = jax.ShapeDtypeStruct((NT, 128), jnp.float32)
    return pl.pallas_call(
        body,
        grid=(NT // BR,),
        in_specs=[
            pl.BlockSpec((BR, 128), lambda i: (i, 0)),
            pl.BlockSpec((128, 256), lambda i: (0, 0)),
            pl.BlockSpec((256, 256), lambda i: (0, 0)),
        ],
        out_specs=[
            pl.BlockSpec((BR, 128), lambda i: (i, 0)),
            pl.BlockSpec((BR, 128), lambda i: (i, 0)),
        ],
        out_shape=[out, out],
    )(s1, w1cat, w2bd)


def _stage_b(s2a, s2b, wca, wcb):
    def body(s2a_ref, s2b_ref, wca_ref, wcb_ref, p_ref):
        p = jnp.dot(jnp.tanh(s2a_ref[...]), wca_ref[...],
                    preferred_element_type=jnp.float32)
        p += jnp.dot(jnp.tanh(s2b_ref[...]), wcb_ref[...],
                     preferred_element_type=jnp.float32)
        p_ref[...] = jnp.concatenate(
            [p, jnp.zeros((BR, 64), jnp.float32)], axis=1)

    return pl.pallas_call(
        body,
        grid=(NT // BR,),
        in_specs=[
            pl.BlockSpec((BR, 128), lambda i: (i, 0)),
            pl.BlockSpec((BR, 128), lambda i: (i, 0)),
            pl.BlockSpec((128, 64), lambda i: (0, 0)),
            pl.BlockSpec((128, 64), lambda i: (0, 0)),
        ],
        out_specs=pl.BlockSpec((BR, 128), lambda i: (i, 0)),
        out_shape=jax.ShapeDtypeStruct((NT, 128), jnp.float32),
    )(s2a, s2b, wca, wcb)


def _stage_c(q, eps_p):
    def body(q_ref, eps_ref, z_ref, m_ref, std_ref):
        s3 = q_ref[...]
        m = s3[:, :32]
        std = jnp.maximum(s3[:, 32:64], 0.0) + 0.0001
        z_ref[...] = eps_ref[...] * std + m
        m_ref[...] = m
        std_ref[...] = std

    out32 = jax.ShapeDtypeStruct((NT, 32), jnp.float32)
    return pl.pallas_call(
        body,
        grid=(NT // BR,),
        in_specs=[
            pl.BlockSpec((BR, 128), lambda i: (i, 0)),
            pl.BlockSpec((BR, 32), lambda i: (i, 0)),
        ],
        out_specs=[
            pl.BlockSpec((BR, 32), lambda i: (i, 0)),
            pl.BlockSpec((BR, 32), lambda i: (i, 0)),
            pl.BlockSpec((BR, 32), lambda i: (i, 0)),
        ],
        out_shape=[out32, out32, out32],
    )(q, eps_p)


def kernel(x, edge_index, W1_0, W1_1, W1_2, W1_3, W2_0, W2_1, W2_2, W2_3,
           Wm, Ws, eps):
    src = edge_index[0]
    dst = edge_index[1]
    pad = EP - E
    src2d = jnp.concatenate(
        [src, jnp.zeros((pad,), jnp.int32)]).reshape(NB, B)
    dla = jnp.where(dst < HALF, dst, DUMMY)
    dlb = jnp.where(dst >= HALF, dst - HALF, DUMMY)
    dpad = jnp.full((pad,), DUMMY, jnp.int32)
    dsta2d = jnp.concatenate([dla, dpad]).reshape(NB, B)
    dstb2d = jnp.concatenate([dlb, dpad]).reshape(NB, B)

    w1cat = jnp.concatenate([W1_0, W1_1, W1_2, W1_3], axis=1)
    z64 = jnp.zeros((64, 64), jnp.float32)
    w2bd = jnp.block([
        [W2_0, z64, z64, z64],
        [z64, W2_1, z64, z64],
        [z64, z64, W2_2, z64],
        [z64, z64, z64, W2_3],
    ])
    wcat = jnp.concatenate([Wm, Ws], axis=1)
    zrows = jnp.zeros((HR, F), jnp.float32)
    eps_p = jnp.concatenate([eps, jnp.zeros((NT - N, 32), jnp.float32)])

    mp_n = _make_mp(N)
    mp_t = _make_mp(NT)

    s1 = mp_n(x, dsta2d, dstb2d, src2d, zrows)
    ga, gb = _stage_a(s1, w1cat, w2bd)

    s2a = mp_t(ga, dsta2d, dstb2d, src2d, zrows)
    s2b = mp_t(gb, dsta2d, dstb2d, src2d, zrows)
    p = _stage_b(s2a, s2b, wcat[:128], wcat[128:])

    q = mp_t(p, dsta2d, dstb2d, src2d, zrows)
    z, m_q_z, std_q_z = _stage_c(q, eps_p)
    return (z[:N], m_q_z[:N], std_q_z[:N])


# speedup vs baseline: 6.9550x; 1.8210x over previous
import functools

import jax
import jax.numpy as jnp
from jax import lax
from jax.experimental import pallas as pl
from jax.experimental.pallas import tpu as pltpu
from jax.experimental.pallas import tpu_sc as plsc

N = 10000
E = 320000
NSUB = 16
NCORE = 2
B = 128
NB = 2560
EP = NB * B
F = 128

NPAD = 10112
DUMMY = N
RPT = NPAD // NSUB


def _make_mp(table_rows, edge_split):
    tpb = NB // (NCORE * NSUB) if edge_split else NB // NSUB
    ch = 8 if edge_split else 16
    nch = tpb // ch
    mesh = plsc.VectorSubcoreMesh(core_axis_name="c", subcore_axis_name="s")
    fs = jax.ShapeDtypeStruct((NPAD, F), jnp.float32)

    @functools.partial(
        pl.kernel,
        out_type=(fs, fs),
        mesh=mesh,
        scratch_types=[
            [pltpu.VMEM((ch, B), jnp.int32) for _ in range(2)],
            [pltpu.VMEM((ch, B), jnp.int32) for _ in range(2)],
            [pltpu.VMEM((B, F), jnp.float32) for _ in range(2)],
            [pltpu.SemaphoreType.DMA for _ in range(2)],
            [pltpu.SemaphoreType.DMA for _ in range(2)],
            [pltpu.SemaphoreType.DMA for _ in range(2)],
            pltpu.VMEM_SHARED((NPAD, F), jnp.float32),
        ],
    )
    def mp(h_a, h_b, src2d, dst2d, zrows, out_a, out_b,
           sib, dib, rows, gsem, ssem, isem, acc):
        c = lax.axis_index("c")
        s = lax.axis_index("s")
        if edge_split:
            tb = (c * NSUB + s) * tpb
        else:
            tb = s * tpb
        r0 = s * RPT
        pltpu.sync_copy(zrows.at[pl.ds(r0, RPT)], acc.at[pl.ds(r0, RPT)])

        def stage(slot, chunk, sem_wait):
            b0 = tb + chunk * ch
            if sem_wait:
                pltpu.async_copy(src2d.at[pl.ds(b0, ch)], sib[slot], isem[slot])
                pltpu.async_copy(dst2d.at[pl.ds(b0, ch)], dib[slot], isem[slot])
            else:
                pltpu.sync_copy(src2d.at[pl.ds(b0, ch)], sib[slot])
                pltpu.sync_copy(dst2d.at[pl.ds(b0, ch)], dib[slot])

        def stage_wait(slot, chunk):
            b0 = tb + chunk * ch
            pltpu.make_async_copy(src2d.at[pl.ds(b0, ch)], sib[slot],
                                  isem[slot]).wait()
            pltpu.make_async_copy(dst2d.at[pl.ds(b0, ch)], dib[slot],
                                  isem[slot]).wait()

        stage(0, 0, False)
        stage(1, 1, True)
        plsc.subcore_barrier()

        def run_chunk(h, slot):
            sb, db = sib[slot], dib[slot]
            pltpu.async_copy(h.at[sb.at[0]], rows[0], gsem[0])
            pltpu.async_copy(h.at[sb.at[1]], rows[1], gsem[1])
            for j in range(ch):
                b = j % 2
                pltpu.make_async_copy(h.at[sb.at[j]], rows[b], gsem[b]).wait()
                pltpu.async_copy(rows[b], acc.at[db.at[j]], ssem[b], add=True)
                if j + 2 < ch:
                    pltpu.make_async_copy(rows[b], acc.at[db.at[j]],
                                          ssem[b]).wait()
                    pltpu.async_copy(h.at[sb.at[j + 2]], rows[b], gsem[b])
            for j in (ch - 2, ch - 1):
                b = j % 2
                pltpu.make_async_copy(rows[b], acc.at[db.at[j]],
                                      ssem[b]).wait()

        def run_all(h):
            def pair(t, carry):
                @pl.when(t > 0)
                def _():
                    stage_wait(0, 2 * t)
                run_chunk(h, 0)

                @pl.when(t < nch // 2 - 1)
                def _():
                    stage(0, 2 * t + 2, True)
                stage_wait(1, 2 * t + 1)
                run_chunk(h, 1)

                @pl.when(t < nch // 2 - 1)
                def _():
                    stage(1, 2 * t + 3, True)
                return carry

            lax.fori_loop(0, nch // 2, pair, 0)

        if edge_split:
            run_all(h_a)
        else:
            for ci, h in ((0, h_a), (1, h_b)):
                @pl.when(c == ci)
                def _():
                    run_all(h)

        plsc.subcore_barrier()
        for ci, out in ((0, out_a), (1, out_b)):
            @pl.when(c == ci)
            def _():
                pltpu.sync_copy(acc.at[pl.ds(r0, RPT)], out.at[pl.ds(r0, RPT)])

    return mp


BR = 1264


def _stage_a(p0, p1, w1cat, w2bd):
    def body(p0_ref, p1_ref, w1_ref, w2_ref, ga_ref, gb_ref):
        s1 = p0_ref[...] + p1_ref[...]
        t = jnp.tanh(jnp.dot(s1, w1_ref[...],
                             preferred_element_type=jnp.float32))
        g = jnp.dot(t, w2_ref[...], preferred_element_type=jnp.float32)
        ga_ref[...] = g[:, :128]
        gb_ref[...] = g[:, 128:]

    out = jax.ShapeDtypeStruct((NPAD, 128), jnp.float32)
    return pl.pallas_call(
        body,
        grid=(NPAD // BR,),
        in_specs=[
            pl.BlockSpec((BR, 128), lambda i: (i, 0)),
            pl.BlockSpec((BR, 128), lambda i: (i, 0)),
            pl.BlockSpec((128, 256), lambda i: (0, 0)),
            pl.BlockSpec((256, 256), lambda i: (0, 0)),
        ],
        out_specs=[
            pl.BlockSpec((BR, 128), lambda i: (i, 0)),
            pl.BlockSpec((BR, 128), lambda i: (i, 0)),
        ],
        out_shape=[out, out],
    )(p0, p1, w1cat, w2bd)


def _stage_b(s2a, s2b, wca, wcb):
    def body(s2a_ref, s2b_ref, wca_ref, wcb_ref, p_ref):
        p = jnp.dot(jnp.tanh(s2a_ref[...]), wca_ref[...],
                    preferred_element_type=jnp.float32)
        p += jnp.dot(jnp.tanh(s2b_ref[...]), wcb_ref[...],
                     preferred_element_type=jnp.float32)
        p_ref[...] = jnp.concatenate(
            [p, jnp.zeros((BR, 64), jnp.float32)], axis=1)

    return pl.pallas_call(
        body,
        grid=(NPAD // BR,),
        in_specs=[
            pl.BlockSpec((BR, 128), lambda i: (i, 0)),
            pl.BlockSpec((BR, 128), lambda i: (i, 0)),
            pl.BlockSpec((128, 64), lambda i: (0, 0)),
            pl.BlockSpec((128, 64), lambda i: (0, 0)),
        ],
        out_specs=pl.BlockSpec((BR, 128), lambda i: (i, 0)),
        out_shape=jax.ShapeDtypeStruct((NPAD, 128), jnp.float32),
    )(s2a, s2b, wca, wcb)


def _stage_c(q0, q1, eps_p):
    def body(q0_ref, q1_ref, eps_ref, z_ref, m_ref, std_ref):
        s3 = q0_ref[...] + q1_ref[...]
        m = s3[:, :32]
        std = jnp.maximum(s3[:, 32:64], 0.0) + 0.0001
        z_ref[...] = eps_ref[...] * std + m
        m_ref[...] = m
        std_ref[...] = std

    out32 = jax.ShapeDtypeStruct((NPAD, 32), jnp.float32)
    return pl.pallas_call(
        body,
        grid=(NPAD // BR,),
        in_specs=[
            pl.BlockSpec((BR, 128), lambda i: (i, 0)),
            pl.BlockSpec((BR, 128), lambda i: (i, 0)),
            pl.BlockSpec((BR, 32), lambda i: (i, 0)),
        ],
        out_specs=[
            pl.BlockSpec((BR, 32), lambda i: (i, 0)),
            pl.BlockSpec((BR, 32), lambda i: (i, 0)),
            pl.BlockSpec((BR, 32), lambda i: (i, 0)),
        ],
        out_shape=[out32, out32, out32],
    )(q0, q1, eps_p)


def kernel(x, edge_index, W1_0, W1_1, W1_2, W1_3, W2_0, W2_1, W2_2, W2_3,
           Wm, Ws, eps):
    src = edge_index[0]
    dst = edge_index[1]
    pad = EP - E
    src2d = jnp.concatenate(
        [src, jnp.zeros((pad,), jnp.int32)]).reshape(NB, B)
    dst2d = jnp.concatenate(
        [dst, jnp.full((pad,), DUMMY, jnp.int32)]).reshape(NB, B)

    w1cat = jnp.concatenate([W1_0, W1_1, W1_2, W1_3], axis=1)
    z64 = jnp.zeros((64, 64), jnp.float32)
    w2bd = jnp.block([
        [W2_0, z64, z64, z64],
        [z64, W2_1, z64, z64],
        [z64, z64, W2_2, z64],
        [z64, z64, z64, W2_3],
    ])
    wcat = jnp.concatenate([Wm, Ws], axis=1)
    zrows = jnp.zeros((NPAD, F), jnp.float32)
    eps_p = jnp.concatenate([eps, jnp.zeros((NPAD - N, 32), jnp.float32)])

    mp_e_n = _make_mp(N, True)
    mp_c = _make_mp(NPAD, False)
    mp_e_t = _make_mp(NPAD, True)

    p0, p1 = mp_e_n(x, x, src2d, dst2d, zrows)
    ga, gb = _stage_a(p0, p1, w1cat, w2bd)

    s2a, s2b = mp_c(ga, gb, src2d, dst2d, zrows)
    p = _stage_b(s2a, s2b, wcat[:128], wcat[128:])

    q0, q1 = mp_e_t(p, p, src2d, dst2d, zrows)
    z, m_q_z, std_q_z = _stage_c(q0, q1, eps_p)
    return (z[:N], m_q_z[:N], std_q_z[:N])


# speedup vs baseline: 22.9788x; 3.3040x over previous
import functools

import jax
import jax.numpy as jnp
from jax import lax
from jax.experimental import pallas as pl
from jax.experimental.pallas import tpu as pltpu
from jax.experimental.pallas import tpu_sc as plsc

N = 10000
E = 320000
NSUB = 16
NCORE = 2
B = 128
NB = 2560
EP = NB * B
F = 128

NPAD = 10112
DUMMY = N
RPT = NPAD // NSUB


def _make_mp(table_rows, edge_split):
    tpb = NB // (NCORE * NSUB) if edge_split else NB // NSUB
    ch = 8 if edge_split else 16
    nch = tpb // ch
    mesh = plsc.VectorSubcoreMesh(core_axis_name="c", subcore_axis_name="s")
    fs = jax.ShapeDtypeStruct((NPAD, F), jnp.float32)

    @functools.partial(
        pl.kernel,
        out_type=(fs, fs),
        mesh=mesh,
        scratch_types=[
            [pltpu.VMEM((ch, B), jnp.int32) for _ in range(2)],
            [pltpu.VMEM((ch, B), jnp.int32) for _ in range(2)],
            [pltpu.VMEM((B, F), jnp.float32) for _ in range(2)],
            [pltpu.SemaphoreType.DMA for _ in range(2)],
            [pltpu.SemaphoreType.DMA for _ in range(2)],
            [pltpu.SemaphoreType.DMA for _ in range(2)],
            pltpu.VMEM_SHARED((NPAD, F), jnp.float32),
        ],
    )
    def mp(h_a, h_b, src2d, dst2d, zrows, out_a, out_b,
           sib, dib, rows, gsem, ssem, isem, acc):
        c = lax.axis_index("c")
        s = lax.axis_index("s")
        if edge_split:
            tb = (c * NSUB + s) * tpb
        else:
            tb = s * tpb
        r0 = s * RPT
        pltpu.sync_copy(zrows.at[pl.ds(r0, RPT)], acc.at[pl.ds(r0, RPT)])

        def stage(slot, chunk, sem_wait):
            b0 = tb + chunk * ch
            if sem_wait:
                pltpu.async_copy(src2d.at[pl.ds(b0, ch)], sib[slot], isem[slot])
                pltpu.async_copy(dst2d.at[pl.ds(b0, ch)], dib[slot], isem[slot])
            else:
                pltpu.sync_copy(src2d.at[pl.ds(b0, ch)], sib[slot])
                pltpu.sync_copy(dst2d.at[pl.ds(b0, ch)], dib[slot])

        def stage_wait(slot, chunk):
            b0 = tb + chunk * ch
            pltpu.make_async_copy(src2d.at[pl.ds(b0, ch)], sib[slot],
                                  isem[slot]).wait()
            pltpu.make_async_copy(dst2d.at[pl.ds(b0, ch)], dib[slot],
                                  isem[slot]).wait()

        stage(0, 0, False)
        stage(1, 1, True)
        plsc.subcore_barrier()

        def run_chunk(h, slot):
            sb, db = sib[slot], dib[slot]
            pltpu.async_copy(h.at[sb.at[0]], rows[0], gsem[0])
            pltpu.async_copy(h.at[sb.at[1]], rows[1], gsem[1])
            for j in range(ch):
                b = j % 2
                pltpu.make_async_copy(h.at[sb.at[j]], rows[b], gsem[b]).wait()
                pltpu.async_copy(rows[b], acc.at[db.at[j]], ssem[b], add=True)
                if j + 2 < ch:
                    pltpu.make_async_copy(rows[b], acc.at[db.at[j]],
                                          ssem[b]).wait()
                    pltpu.async_copy(h.at[sb.at[j + 2]], rows[b], gsem[b])
            for j in (ch - 2, ch - 1):
                b = j % 2
                pltpu.make_async_copy(rows[b], acc.at[db.at[j]],
                                      ssem[b]).wait()

        def run_all(h):
            def pair(t, carry):
                @pl.when(t > 0)
                def _():
                    stage_wait(0, 2 * t)
                run_chunk(h, 0)

                @pl.when(t < nch // 2 - 1)
                def _():
                    stage(0, 2 * t + 2, True)
                stage_wait(1, 2 * t + 1)
                run_chunk(h, 1)

                @pl.when(t < nch // 2 - 1)
                def _():
                    stage(1, 2 * t + 3, True)
                return carry

            lax.fori_loop(0, nch // 2, pair, 0)

        if edge_split:
            run_all(h_a)
        else:
            for ci, h in ((0, h_a), (1, h_b)):
                @pl.when(c == ci)
                def _():
                    run_all(h)

        plsc.subcore_barrier()
        for ci, out in ((0, out_a), (1, out_b)):
            @pl.when(c == ci)
            def _():
                pltpu.sync_copy(acc.at[pl.ds(r0, RPT)], out.at[pl.ds(r0, RPT)])

    return mp


BR = 1264


def _stage_a(p0, p1, w1cat, w2bd):
    def body(p0_ref, p1_ref, w1_ref, w2_ref, ga_ref, gb_ref):
        s1 = p0_ref[...] + p1_ref[...]
        t = jnp.tanh(jnp.dot(s1, w1_ref[...],
                             preferred_element_type=jnp.float32))
        g = jnp.dot(t, w2_ref[...], preferred_element_type=jnp.float32)
        ga_ref[...] = g[:, :128]
        gb_ref[...] = g[:, 128:]

    out = jax.ShapeDtypeStruct((NPAD, 128), jnp.float32)
    return pl.pallas_call(
        body,
        grid=(NPAD // BR,),
        in_specs=[
            pl.BlockSpec((BR, 128), lambda i: (i, 0)),
            pl.BlockSpec((BR, 128), lambda i: (i, 0)),
            pl.BlockSpec((128, 256), lambda i: (0, 0)),
            pl.BlockSpec((256, 256), lambda i: (0, 0)),
        ],
        out_specs=[
            pl.BlockSpec((BR, 128), lambda i: (i, 0)),
            pl.BlockSpec((BR, 128), lambda i: (i, 0)),
        ],
        out_shape=[out, out],
    )(p0, p1, w1cat, w2bd)


def _stage_b(s2a, s2b, wca, wcb):
    def body(s2a_ref, s2b_ref, wca_ref, wcb_ref, p_ref):
        p = jnp.dot(jnp.tanh(s2a_ref[...]), wca_ref[...],
                    preferred_element_type=jnp.float32)
        p += jnp.dot(jnp.tanh(s2b_ref[...]), wcb_ref[...],
                     preferred_element_type=jnp.float32)
        p_ref[...] = jnp.concatenate(
            [p, jnp.zeros((BR, 64), jnp.float32)], axis=1)

    return pl.pallas_call(
        body,
        grid=(NPAD // BR,),
        in_specs=[
            pl.BlockSpec((BR, 128), lambda i: (i, 0)),
            pl.BlockSpec((BR, 128), lambda i: (i, 0)),
            pl.BlockSpec((128, 64), lambda i: (0, 0)),
            pl.BlockSpec((128, 64), lambda i: (0, 0)),
        ],
        out_specs=pl.BlockSpec((BR, 128), lambda i: (i, 0)),
        out_shape=jax.ShapeDtypeStruct((NPAD, 128), jnp.float32),
    )(s2a, s2b, wca, wcb)


def _stage_c(q0, q1, eps_p):
    def body(q0_ref, q1_ref, eps_ref, z_ref, m_ref, std_ref):
        s3 = q0_ref[...] + q1_ref[...]
        m = s3[:, :32]
        std = jnp.maximum(s3[:, 32:64], 0.0) + 0.0001
        z_ref[...] = eps_ref[...] * std + m
        m_ref[...] = m
        std_ref[...] = std

    out32 = jax.ShapeDtypeStruct((NPAD, 32), jnp.float32)
    return pl.pallas_call(
        body,
        grid=(NPAD // BR,),
        in_specs=[
            pl.BlockSpec((BR, 128), lambda i: (i, 0)),
            pl.BlockSpec((BR, 128), lambda i: (i, 0)),
            pl.BlockSpec((BR, 32), lambda i: (i, 0)),
        ],
        out_specs=[
            pl.BlockSpec((BR, 32), lambda i: (i, 0)),
            pl.BlockSpec((BR, 32), lambda i: (i, 0)),
            pl.BlockSpec((BR, 32), lambda i: (i, 0)),
        ],
        out_shape=[out32, out32, out32],
    )(q0, q1, eps_p)


def kernel(x, edge_index, W1_0, W1_1, W1_2, W1_3, W2_0, W2_1, W2_2, W2_3,
           Wm, Ws, eps):
    src = edge_index[0]
    dst = edge_index[1]
    pad = EP - E
    pad_i = jnp.arange(pad, dtype=jnp.int32)
    src2d = jnp.concatenate([src, pad_i % N]).reshape(NB, B)
    dst2d = jnp.concatenate(
        [dst, DUMMY + pad_i % (NPAD - N)]).reshape(NB, B)

    w1cat = jnp.concatenate([W1_0, W1_1, W1_2, W1_3], axis=1)
    z64 = jnp.zeros((64, 64), jnp.float32)
    w2bd = jnp.block([
        [W2_0, z64, z64, z64],
        [z64, W2_1, z64, z64],
        [z64, z64, W2_2, z64],
        [z64, z64, z64, W2_3],
    ])
    wcat = jnp.concatenate([Wm, Ws], axis=1)
    zrows = jnp.zeros((NPAD, F), jnp.float32)
    eps_p = jnp.concatenate([eps, jnp.zeros((NPAD - N, 32), jnp.float32)])

    mp_e_n = _make_mp(N, True)
    mp_c = _make_mp(NPAD, False)
    mp_e_t = _make_mp(NPAD, True)

    p0, p1 = mp_e_n(x, x, src2d, dst2d, zrows)
    ga, gb = _stage_a(p0, p1, w1cat, w2bd)

    s2a, s2b = mp_c(ga, gb, src2d, dst2d, zrows)
    p = _stage_b(s2a, s2b, wcat[:128], wcat[128:])

    q0, q1 = mp_e_t(p, p, src2d, dst2d, zrows)
    z, m_q_z, std_q_z = _stage_c(q0, q1, eps_p)
    return (z[:N], m_q_z[:N], std_q_z[:N])


# speedup vs baseline: 24.4255x; 1.0630x over previous
import functools

import jax
import jax.numpy as jnp
from jax import lax
from jax.experimental import pallas as pl
from jax.experimental.pallas import tpu as pltpu
from jax.experimental.pallas import tpu_sc as plsc

N = 10000
E = 320000
NSUB = 16
NCORE = 2
B = 128
NB = 2560
EP = NB * B
F = 128

NPAD = 10112
DUMMY = N
RPT = NPAD // NSUB


def _make_mp(table_rows, edge_split, w=F, tc_tiling=True):
    tpb = NB // (NCORE * NSUB) if edge_split else NB // NSUB
    ch = 8 if edge_split else 16
    nch = tpb // ch
    mesh = plsc.VectorSubcoreMesh(core_axis_name="c", subcore_axis_name="s")
    fs = jax.ShapeDtypeStruct((NPAD, w), jnp.float32)

    @functools.partial(
        pl.kernel,
        out_type=(fs, fs),
        mesh=mesh,
        compiler_params=pltpu.CompilerParams(use_tc_tiling_on_sc=tc_tiling),
        scratch_types=[
            [pltpu.VMEM((ch, B), jnp.int32) for _ in range(2)],
            [pltpu.VMEM((ch, B), jnp.int32) for _ in range(2)],
            [pltpu.VMEM((B, w), jnp.float32) for _ in range(2)],
            [pltpu.SemaphoreType.DMA for _ in range(2)],
            [pltpu.SemaphoreType.DMA for _ in range(2)],
            [pltpu.SemaphoreType.DMA for _ in range(2)],
            pltpu.VMEM_SHARED((NPAD, w), jnp.float32),
        ],
    )
    def mp(h_a, h_b, src2d, dst2d, zrows, out_a, out_b,
           sib, dib, rows, gsem, ssem, isem, acc):
        c = lax.axis_index("c")
        s = lax.axis_index("s")
        if edge_split:
            tb = (c * NSUB + s) * tpb
        else:
            tb = s * tpb
        r0 = s * RPT
        pltpu.sync_copy(zrows.at[pl.ds(r0, RPT)], acc.at[pl.ds(r0, RPT)])

        def stage(slot, chunk, sem_wait):
            b0 = tb + chunk * ch
            if sem_wait:
                pltpu.async_copy(src2d.at[pl.ds(b0, ch)], sib[slot], isem[slot])
                pltpu.async_copy(dst2d.at[pl.ds(b0, ch)], dib[slot], isem[slot])
            else:
                pltpu.sync_copy(src2d.at[pl.ds(b0, ch)], sib[slot])
                pltpu.sync_copy(dst2d.at[pl.ds(b0, ch)], dib[slot])

        def stage_wait(slot, chunk):
            b0 = tb + chunk * ch
            pltpu.make_async_copy(src2d.at[pl.ds(b0, ch)], sib[slot],
                                  isem[slot]).wait()
            pltpu.make_async_copy(dst2d.at[pl.ds(b0, ch)], dib[slot],
                                  isem[slot]).wait()

        stage(0, 0, False)
        stage(1, 1, True)
        plsc.subcore_barrier()

        def run_chunk(h, slot):
            sb, db = sib[slot], dib[slot]
            pltpu.async_copy(h.at[sb.at[0]], rows[0], gsem[0])
            pltpu.async_copy(h.at[sb.at[1]], rows[1], gsem[1])
            for j in range(ch):
                b = j % 2
                pltpu.make_async_copy(h.at[sb.at[j]], rows[b], gsem[b]).wait()
                pltpu.async_copy(rows[b], acc.at[db.at[j]], ssem[b], add=True)
                if j + 2 < ch:
                    pltpu.make_async_copy(rows[b], acc.at[db.at[j]],
                                          ssem[b]).wait()
                    pltpu.async_copy(h.at[sb.at[j + 2]], rows[b], gsem[b])
            for j in (ch - 2, ch - 1):
                b = j % 2
                pltpu.make_async_copy(rows[b], acc.at[db.at[j]],
                                      ssem[b]).wait()

        def run_all(h):
            def pair(t, carry):
                @pl.when(t > 0)
                def _():
                    stage_wait(0, 2 * t)
                run_chunk(h, 0)

                @pl.when(t < nch // 2 - 1)
                def _():
                    stage(0, 2 * t + 2, True)
                stage_wait(1, 2 * t + 1)
                run_chunk(h, 1)

                @pl.when(t < nch // 2 - 1)
                def _():
                    stage(1, 2 * t + 3, True)
                return carry

            lax.fori_loop(0, nch // 2, pair, 0)

        if edge_split:
            run_all(h_a)
        else:
            for ci, h in ((0, h_a), (1, h_b)):
                @pl.when(c == ci)
                def _():
                    run_all(h)

        plsc.subcore_barrier()
        for ci, out in ((0, out_a), (1, out_b)):
            @pl.when(c == ci)
            def _():
                pltpu.sync_copy(acc.at[pl.ds(r0, RPT)], out.at[pl.ds(r0, RPT)])

    return mp


BR = 1264


def _stage_a(p0, p1, w1cat, w2bd):
    def body(p0_ref, p1_ref, w1_ref, w2_ref, ga_ref, gb_ref):
        s1 = p0_ref[...] + p1_ref[...]
        t = jnp.tanh(jnp.dot(s1, w1_ref[...],
                             preferred_element_type=jnp.float32))
        g = jnp.dot(t, w2_ref[...], preferred_element_type=jnp.float32)
        ga_ref[...] = g[:, :128]
        gb_ref[...] = g[:, 128:]

    out = jax.ShapeDtypeStruct((NPAD, 128), jnp.float32)
    return pl.pallas_call(
        body,
        grid=(NPAD // BR,),
        in_specs=[
            pl.BlockSpec((BR, 128), lambda i: (i, 0)),
            pl.BlockSpec((BR, 128), lambda i: (i, 0)),
            pl.BlockSpec((128, 256), lambda i: (0, 0)),
            pl.BlockSpec((256, 256), lambda i: (0, 0)),
        ],
        out_specs=[
            pl.BlockSpec((BR, 128), lambda i: (i, 0)),
            pl.BlockSpec((BR, 128), lambda i: (i, 0)),
        ],
        out_shape=[out, out],
    )(p0, p1, w1cat, w2bd)


def _stage_b(s2a, s2b, wca, wcb):
    def body(s2a_ref, s2b_ref, wca_ref, wcb_ref, p_ref):
        p = jnp.dot(jnp.tanh(s2a_ref[...]), wca_ref[...],
                    preferred_element_type=jnp.float32)
        p += jnp.dot(jnp.tanh(s2b_ref[...]), wcb_ref[...],
                     preferred_element_type=jnp.float32)
        p_ref[...] = p

    return pl.pallas_call(
        body,
        grid=(NPAD // BR,),
        in_specs=[
            pl.BlockSpec((BR, 128), lambda i: (i, 0)),
            pl.BlockSpec((BR, 128), lambda i: (i, 0)),
            pl.BlockSpec((128, 64), lambda i: (0, 0)),
            pl.BlockSpec((128, 64), lambda i: (0, 0)),
        ],
        out_specs=pl.BlockSpec((BR, 64), lambda i: (i, 0)),
        out_shape=jax.ShapeDtypeStruct((NPAD, 64), jnp.float32),
    )(s2a, s2b, wca, wcb)


def _stage_c(q0, q1, eps_p):
    def body(q0_ref, q1_ref, eps_ref, z_ref, m_ref, std_ref):
        s3 = q0_ref[...] + q1_ref[...]
        m = s3[:, :32]
        std = jnp.maximum(s3[:, 32:64], 0.0) + 0.0001
        z_ref[...] = eps_ref[...] * std + m
        m_ref[...] = m
        std_ref[...] = std

    out32 = jax.ShapeDtypeStruct((NPAD, 32), jnp.float32)
    return pl.pallas_call(
        body,
        grid=(NPAD // BR,),
        in_specs=[
            pl.BlockSpec((BR, 64), lambda i: (i, 0)),
            pl.BlockSpec((BR, 64), lambda i: (i, 0)),
            pl.BlockSpec((BR, 32), lambda i: (i, 0)),
        ],
        out_specs=[
            pl.BlockSpec((BR, 32), lambda i: (i, 0)),
            pl.BlockSpec((BR, 32), lambda i: (i, 0)),
            pl.BlockSpec((BR, 32), lambda i: (i, 0)),
        ],
        out_shape=[out32, out32, out32],
    )(q0, q1, eps_p)


def kernel(x, edge_index, W1_0, W1_1, W1_2, W1_3, W2_0, W2_1, W2_2, W2_3,
           Wm, Ws, eps):
    src = edge_index[0]
    dst = edge_index[1]
    pad = EP - E
    pad_i = jnp.arange(pad, dtype=jnp.int32)
    src2d = jnp.concatenate([src, pad_i % N]).reshape(NB, B)
    dst2d = jnp.concatenate(
        [dst, DUMMY + pad_i % (NPAD - N)]).reshape(NB, B)

    w1cat = jnp.concatenate([W1_0, W1_1, W1_2, W1_3], axis=1)
    z64 = jnp.zeros((64, 64), jnp.float32)
    w2bd = jnp.block([
        [W2_0, z64, z64, z64],
        [z64, W2_1, z64, z64],
        [z64, z64, W2_2, z64],
        [z64, z64, z64, W2_3],
    ])
    wcat = jnp.concatenate([Wm, Ws], axis=1)
    zrows = jnp.zeros((NPAD, F), jnp.float32)
    zrows64 = jnp.zeros((NPAD, 64), jnp.float32)
    eps_p = jnp.concatenate([eps, jnp.zeros((NPAD - N, 32), jnp.float32)])

    mp_e_n = _make_mp(N, True)
    mp_c = _make_mp(NPAD, False)
    mp_e64 = _make_mp(NPAD, True, 64, False)

    p0, p1 = mp_e_n(x, x, src2d, dst2d, zrows)
    ga, gb = _stage_a(p0, p1, w1cat, w2bd)

    s2a, s2b = mp_c(ga, gb, src2d, dst2d, zrows)
    p = _stage_b(s2a, s2b, wcat[:128], wcat[128:])

    q0, q1 = mp_e64(p, p, src2d, dst2d, zrows64)
    z, m_q_z, std_q_z = _stage_c(q0, q1, eps_p)
    return (z[:N], m_q_z[:N], std_q_z[:N])


# speedup vs baseline: 24.8589x; 1.0177x over previous
import functools

import jax
import jax.numpy as jnp
from jax import lax
from jax.experimental import pallas as pl
from jax.experimental.pallas import tpu as pltpu
from jax.experimental.pallas import tpu_sc as plsc

N = 10000
E = 320000
NSUB = 16
NCORE = 2
B = 128
NB = 2560
EP = NB * B
F = 128

NPAD = 10112
DUMMY = N
RPT = NPAD // NSUB


def _make_mp(table_rows, edge_split, w=F, tc_tiling=True):
    tpb = NB // (NCORE * NSUB) if edge_split else NB // NSUB
    ch = 8 if edge_split else 16
    nch = tpb // ch
    nbuf = 2 if w == F else 4
    mesh = plsc.VectorSubcoreMesh(core_axis_name="c", subcore_axis_name="s")
    fs = jax.ShapeDtypeStruct((NPAD, w), jnp.float32)

    @functools.partial(
        pl.kernel,
        out_type=(fs, fs),
        mesh=mesh,
        compiler_params=pltpu.CompilerParams(use_tc_tiling_on_sc=tc_tiling),
        scratch_types=[
            [pltpu.VMEM((ch, B), jnp.int32) for _ in range(2)],
            [pltpu.VMEM((ch, B), jnp.int32) for _ in range(2)],
            [pltpu.VMEM((B, w), jnp.float32) for _ in range(nbuf)],
            [pltpu.SemaphoreType.DMA for _ in range(nbuf)],
            [pltpu.SemaphoreType.DMA for _ in range(nbuf)],
            [pltpu.SemaphoreType.DMA for _ in range(2)],
            pltpu.VMEM_SHARED((NPAD, w), jnp.float32),
        ],
    )
    def mp(h_a, h_b, src2d, dst2d, zrows, out_a, out_b,
           sib, dib, rows, gsem, ssem, isem, acc):
        c = lax.axis_index("c")
        s = lax.axis_index("s")
        if edge_split:
            tb = (c * NSUB + s) * tpb
        else:
            tb = s * tpb
        r0 = s * RPT
        pltpu.sync_copy(zrows.at[pl.ds(r0, RPT)], acc.at[pl.ds(r0, RPT)])

        def stage(slot, chunk, sem_wait):
            b0 = tb + chunk * ch
            if sem_wait:
                pltpu.async_copy(src2d.at[pl.ds(b0, ch)], sib[slot], isem[slot])
                pltpu.async_copy(dst2d.at[pl.ds(b0, ch)], dib[slot], isem[slot])
            else:
                pltpu.sync_copy(src2d.at[pl.ds(b0, ch)], sib[slot])
                pltpu.sync_copy(dst2d.at[pl.ds(b0, ch)], dib[slot])

        def stage_wait(slot, chunk):
            b0 = tb + chunk * ch
            pltpu.make_async_copy(src2d.at[pl.ds(b0, ch)], sib[slot],
                                  isem[slot]).wait()
            pltpu.make_async_copy(dst2d.at[pl.ds(b0, ch)], dib[slot],
                                  isem[slot]).wait()

        stage(0, 0, False)
        stage(1, 1, True)
        plsc.subcore_barrier()

        def run_chunk(h, slot):
            sb, db = sib[slot], dib[slot]
            for b in range(nbuf):
                pltpu.async_copy(h.at[sb.at[b]], rows[b], gsem[b])
            for j in range(ch):
                b = j % nbuf
                pltpu.make_async_copy(h.at[sb.at[j]], rows[b], gsem[b]).wait()
                pltpu.async_copy(rows[b], acc.at[db.at[j]], ssem[b], add=True)
                if j + nbuf < ch:
                    pltpu.make_async_copy(rows[b], acc.at[db.at[j]],
                                          ssem[b]).wait()
                    pltpu.async_copy(h.at[sb.at[j + nbuf]], rows[b], gsem[b])
            for j in range(ch - nbuf, ch):
                b = j % nbuf
                pltpu.make_async_copy(rows[b], acc.at[db.at[j]],
                                      ssem[b]).wait()

        def run_all(h):
            def pair(t, carry):
                @pl.when(t > 0)
                def _():
                    stage_wait(0, 2 * t)
                run_chunk(h, 0)

                @pl.when(t < nch // 2 - 1)
                def _():
                    stage(0, 2 * t + 2, True)
                stage_wait(1, 2 * t + 1)
                run_chunk(h, 1)

                @pl.when(t < nch // 2 - 1)
                def _():
                    stage(1, 2 * t + 3, True)
                return carry

            lax.fori_loop(0, nch // 2, pair, 0)

        if edge_split:
            run_all(h_a)
        else:
            for ci, h in ((0, h_a), (1, h_b)):
                @pl.when(c == ci)
                def _():
                    run_all(h)

        plsc.subcore_barrier()
        for ci, out in ((0, out_a), (1, out_b)):
            @pl.when(c == ci)
            def _():
                pltpu.sync_copy(acc.at[pl.ds(r0, RPT)], out.at[pl.ds(r0, RPT)])

    return mp


BR = 1264


def _stage_a(p0, p1, w1cat, w2bd):
    def body(p0_ref, p1_ref, w1_ref, w2_ref, ga_ref, gb_ref):
        s1 = p0_ref[...] + p1_ref[...]
        t = jnp.tanh(jnp.dot(s1, w1_ref[...],
                             preferred_element_type=jnp.float32))
        g = jnp.dot(t, w2_ref[...], preferred_element_type=jnp.float32)
        ga_ref[...] = g[:, :128]
        gb_ref[...] = g[:, 128:]

    out = jax.ShapeDtypeStruct((NPAD, 128), jnp.float32)
    return pl.pallas_call(
        body,
        grid=(NPAD // BR,),
        in_specs=[
            pl.BlockSpec((BR, 128), lambda i: (i, 0)),
            pl.BlockSpec((BR, 128), lambda i: (i, 0)),
            pl.BlockSpec((128, 256), lambda i: (0, 0)),
            pl.BlockSpec((256, 256), lambda i: (0, 0)),
        ],
        out_specs=[
            pl.BlockSpec((BR, 128), lambda i: (i, 0)),
            pl.BlockSpec((BR, 128), lambda i: (i, 0)),
        ],
        out_shape=[out, out],
    )(p0, p1, w1cat, w2bd)


def _stage_b(s2a, s2b, wca, wcb):
    def body(s2a_ref, s2b_ref, wca_ref, wcb_ref, p_ref):
        p = jnp.dot(jnp.tanh(s2a_ref[...]), wca_ref[...],
                    preferred_element_type=jnp.float32)
        p += jnp.dot(jnp.tanh(s2b_ref[...]), wcb_ref[...],
                     preferred_element_type=jnp.float32)
        p_ref[...] = p

    return pl.pallas_call(
        body,
        grid=(NPAD // BR,),
        in_specs=[
            pl.BlockSpec((BR, 128), lambda i: (i, 0)),
            pl.BlockSpec((BR, 128), lambda i: (i, 0)),
            pl.BlockSpec((128, 64), lambda i: (0, 0)),
            pl.BlockSpec((128, 64), lambda i: (0, 0)),
        ],
        out_specs=pl.BlockSpec((BR, 64), lambda i: (i, 0)),
        out_shape=jax.ShapeDtypeStruct((NPAD, 64), jnp.float32),
    )(s2a, s2b, wca, wcb)


def _stage_c(q0, q1, eps_p):
    def body(q0_ref, q1_ref, eps_ref, z_ref, m_ref, std_ref):
        s3 = q0_ref[...] + q1_ref[...]
        m = s3[:, :32]
        std = jnp.maximum(s3[:, 32:64], 0.0) + 0.0001
        z_ref[...] = eps_ref[...] * std + m
        m_ref[...] = m
        std_ref[...] = std

    out32 = jax.ShapeDtypeStruct((NPAD, 32), jnp.float32)
    return pl.pallas_call(
        body,
        grid=(NPAD // BR,),
        in_specs=[
            pl.BlockSpec((BR, 64), lambda i: (i, 0)),
            pl.BlockSpec((BR, 64), lambda i: (i, 0)),
            pl.BlockSpec((BR, 32), lambda i: (i, 0)),
        ],
        out_specs=[
            pl.BlockSpec((BR, 32), lambda i: (i, 0)),
            pl.BlockSpec((BR, 32), lambda i: (i, 0)),
            pl.BlockSpec((BR, 32), lambda i: (i, 0)),
        ],
        out_shape=[out32, out32, out32],
    )(q0, q1, eps_p)


def kernel(x, edge_index, W1_0, W1_1, W1_2, W1_3, W2_0, W2_1, W2_2, W2_3,
           Wm, Ws, eps):
    src = edge_index[0]
    dst = edge_index[1]
    pad = EP - E
    pad_i = jnp.arange(pad, dtype=jnp.int32)
    src2d = jnp.concatenate([src, pad_i % N]).reshape(NB, B)
    dst2d = jnp.concatenate(
        [dst, DUMMY + pad_i % (NPAD - N)]).reshape(NB, B)

    w1cat = jnp.concatenate([W1_0, W1_1, W1_2, W1_3], axis=1)
    z64 = jnp.zeros((64, 64), jnp.float32)
    w2bd = jnp.block([
        [W2_0, z64, z64, z64],
        [z64, W2_1, z64, z64],
        [z64, z64, W2_2, z64],
        [z64, z64, z64, W2_3],
    ])
    wcat = jnp.concatenate([Wm, Ws], axis=1)
    zrows = jnp.zeros((NPAD, F), jnp.float32)
    zrows64 = jnp.zeros((NPAD, 64), jnp.float32)
    eps_p = jnp.concatenate([eps, jnp.zeros((NPAD - N, 32), jnp.float32)])

    mp_e_n = _make_mp(N, True)
    mp_c = _make_mp(NPAD, False)
    mp_e64 = _make_mp(NPAD, True, 64, False)

    p0, p1 = mp_e_n(x, x, src2d, dst2d, zrows)
    ga, gb = _stage_a(p0, p1, w1cat, w2bd)

    s2a, s2b = mp_c(ga, gb, src2d, dst2d, zrows)
    p = _stage_b(s2a, s2b, wcat[:128], wcat[128:])

    q0, q1 = mp_e64(p, p, src2d, dst2d, zrows64)
    z, m_q_z, std_q_z = _stage_c(q0, q1, eps_p)
    return (z[:N], m_q_z[:N], std_q_z[:N])


# speedup vs baseline: 25.5072x; 1.0261x over previous
import functools

import jax
import jax.numpy as jnp
from jax import lax
from jax.experimental import pallas as pl
from jax.experimental.pallas import tpu as pltpu
from jax.experimental.pallas import tpu_sc as plsc

N = 10000
E = 320000
NSUB = 16
NCORE = 2
B = 128
NB = 2560
EP = NB * B
F = 128

NPAD = 10112
DUMMY = N
RPT = NPAD // NSUB


def _make_mp(table_rows, edge_split, w=F, tc_tiling=True):
    tpb = NB // (NCORE * NSUB) if edge_split else NB // NSUB
    ch = 16
    nch = tpb // ch
    nbuf = 2 if w == F else 4
    mesh = plsc.VectorSubcoreMesh(core_axis_name="c", subcore_axis_name="s")
    fs = jax.ShapeDtypeStruct((NPAD, w), jnp.float32)

    @functools.partial(
        pl.kernel,
        out_type=(fs, fs),
        mesh=mesh,
        compiler_params=pltpu.CompilerParams(use_tc_tiling_on_sc=tc_tiling),
        scratch_types=[
            [pltpu.VMEM((ch, B), jnp.int32) for _ in range(2)],
            [pltpu.VMEM((ch, B), jnp.int32) for _ in range(2)],
            [pltpu.VMEM((B, w), jnp.float32) for _ in range(nbuf)],
            [pltpu.SemaphoreType.DMA for _ in range(nbuf)],
            [pltpu.SemaphoreType.DMA for _ in range(nbuf)],
            [pltpu.SemaphoreType.DMA for _ in range(2)],
            pltpu.VMEM_SHARED((NPAD, w), jnp.float32),
        ],
    )
    def mp(h_a, h_b, src2d, dst2d, zrows, out_a, out_b,
           sib, dib, rows, gsem, ssem, isem, acc):
        c = lax.axis_index("c")
        s = lax.axis_index("s")
        if edge_split:
            tb = (c * NSUB + s) * tpb
        else:
            tb = s * tpb
        r0 = s * RPT
        pltpu.sync_copy(zrows.at[pl.ds(r0, RPT)], acc.at[pl.ds(r0, RPT)])

        def stage(slot, chunk, sem_wait):
            b0 = tb + chunk * ch
            if sem_wait:
                pltpu.async_copy(src2d.at[pl.ds(b0, ch)], sib[slot], isem[slot])
                pltpu.async_copy(dst2d.at[pl.ds(b0, ch)], dib[slot], isem[slot])
            else:
                pltpu.sync_copy(src2d.at[pl.ds(b0, ch)], sib[slot])
                pltpu.sync_copy(dst2d.at[pl.ds(b0, ch)], dib[slot])

        def stage_wait(slot, chunk):
            b0 = tb + chunk * ch
            pltpu.make_async_copy(src2d.at[pl.ds(b0, ch)], sib[slot],
                                  isem[slot]).wait()
            pltpu.make_async_copy(dst2d.at[pl.ds(b0, ch)], dib[slot],
                                  isem[slot]).wait()

        stage(0, 0, False)
        stage(1, 1, True)
        plsc.subcore_barrier()

        def run_chunk(h, slot):
            sb, db = sib[slot], dib[slot]
            for b in range(nbuf):
                pltpu.async_copy(h.at[sb.at[b]], rows[b], gsem[b])
            for j in range(ch):
                b = j % nbuf
                pltpu.make_async_copy(h.at[sb.at[j]], rows[b], gsem[b]).wait()
                pltpu.async_copy(rows[b], acc.at[db.at[j]], ssem[b], add=True)
                if j + nbuf < ch:
                    pltpu.make_async_copy(rows[b], acc.at[db.at[j]],
                                          ssem[b]).wait()
                    pltpu.async_copy(h.at[sb.at[j + nbuf]], rows[b], gsem[b])
            for j in range(ch - nbuf, ch):
                b = j % nbuf
                pltpu.make_async_copy(rows[b], acc.at[db.at[j]],
                                      ssem[b]).wait()

        def run_all(h):
            def pair(t, carry):
                @pl.when(t > 0)
                def _():
                    stage_wait(0, 2 * t)
                run_chunk(h, 0)

                @pl.when(2 * t + 2 < nch)
                def _():
                    stage(0, 2 * t + 2, True)
                stage_wait(1, 2 * t + 1)
                run_chunk(h, 1)

                @pl.when(2 * t + 3 < nch)
                def _():
                    stage(1, 2 * t + 3, True)
                return carry

            lax.fori_loop(0, nch // 2, pair, 0)
            if nch % 2:
                stage_wait(0, nch - 1)
                run_chunk(h, 0)

        if edge_split:
            run_all(h_a)
        else:
            for ci, h in ((0, h_a), (1, h_b)):
                @pl.when(c == ci)
                def _():
                    run_all(h)

        plsc.subcore_barrier()
        for ci, out in ((0, out_a), (1, out_b)):
            @pl.when(c == ci)
            def _():
                pltpu.sync_copy(acc.at[pl.ds(r0, RPT)], out.at[pl.ds(r0, RPT)])

    return mp


BR = 1264


def _stage_a(p0, p1, w1cat, w2bd):
    def body(p0_ref, p1_ref, w1_ref, w2_ref, ga_ref, gb_ref):
        s1 = p0_ref[...] + p1_ref[...]
        t = jnp.tanh(jnp.dot(s1, w1_ref[...],
                             preferred_element_type=jnp.float32))
        g = jnp.dot(t, w2_ref[...], preferred_element_type=jnp.float32)
        ga_ref[...] = g[:, :128]
        gb_ref[...] = g[:, 128:]

    out = jax.ShapeDtypeStruct((NPAD, 128), jnp.float32)
    return pl.pallas_call(
        body,
        grid=(NPAD // BR,),
        in_specs=[
            pl.BlockSpec((BR, 128), lambda i: (i, 0)),
            pl.BlockSpec((BR, 128), lambda i: (i, 0)),
            pl.BlockSpec((128, 256), lambda i: (0, 0)),
            pl.BlockSpec((256, 256), lambda i: (0, 0)),
        ],
        out_specs=[
            pl.BlockSpec((BR, 128), lambda i: (i, 0)),
            pl.BlockSpec((BR, 128), lambda i: (i, 0)),
        ],
        out_shape=[out, out],
    )(p0, p1, w1cat, w2bd)


def _stage_b(s2a, s2b, wca, wcb):
    def body(s2a_ref, s2b_ref, wca_ref, wcb_ref, p_ref):
        p = jnp.dot(jnp.tanh(s2a_ref[...]), wca_ref[...],
                    preferred_element_type=jnp.float32)
        p += jnp.dot(jnp.tanh(s2b_ref[...]), wcb_ref[...],
                     preferred_element_type=jnp.float32)
        p_ref[...] = p

    return pl.pallas_call(
        body,
        grid=(NPAD // BR,),
        in_specs=[
            pl.BlockSpec((BR, 128), lambda i: (i, 0)),
            pl.BlockSpec((BR, 128), lambda i: (i, 0)),
            pl.BlockSpec((128, 64), lambda i: (0, 0)),
            pl.BlockSpec((128, 64), lambda i: (0, 0)),
        ],
        out_specs=pl.BlockSpec((BR, 64), lambda i: (i, 0)),
        out_shape=jax.ShapeDtypeStruct((NPAD, 64), jnp.float32),
    )(s2a, s2b, wca, wcb)


def _stage_c(q0, q1, eps_p):
    def body(q0_ref, q1_ref, eps_ref, z_ref, m_ref, std_ref):
        s3 = q0_ref[...] + q1_ref[...]
        m = s3[:, :32]
        std = jnp.maximum(s3[:, 32:64], 0.0) + 0.0001
        z_ref[...] = eps_ref[...] * std + m
        m_ref[...] = m
        std_ref[...] = std

    out32 = jax.ShapeDtypeStruct((NPAD, 32), jnp.float32)
    return pl.pallas_call(
        body,
        grid=(NPAD // BR,),
        in_specs=[
            pl.BlockSpec((BR, 64), lambda i: (i, 0)),
            pl.BlockSpec((BR, 64), lambda i: (i, 0)),
            pl.BlockSpec((BR, 32), lambda i: (i, 0)),
        ],
        out_specs=[
            pl.BlockSpec((BR, 32), lambda i: (i, 0)),
            pl.BlockSpec((BR, 32), lambda i: (i, 0)),
            pl.BlockSpec((BR, 32), lambda i: (i, 0)),
        ],
        out_shape=[out32, out32, out32],
    )(q0, q1, eps_p)


def kernel(x, edge_index, W1_0, W1_1, W1_2, W1_3, W2_0, W2_1, W2_2, W2_3,
           Wm, Ws, eps):
    src = edge_index[0]
    dst = edge_index[1]
    pad = EP - E
    pad_i = jnp.arange(pad, dtype=jnp.int32)
    src2d = jnp.concatenate([src, pad_i % N]).reshape(NB, B)
    dst2d = jnp.concatenate(
        [dst, DUMMY + pad_i % (NPAD - N)]).reshape(NB, B)

    w1cat = jnp.concatenate([W1_0, W1_1, W1_2, W1_3], axis=1)
    z64 = jnp.zeros((64, 64), jnp.float32)
    w2bd = jnp.block([
        [W2_0, z64, z64, z64],
        [z64, W2_1, z64, z64],
        [z64, z64, W2_2, z64],
        [z64, z64, z64, W2_3],
    ])
    wcat = jnp.concatenate([Wm, Ws], axis=1)
    zrows = jnp.zeros((NPAD, F), jnp.float32)
    zrows64 = jnp.zeros((NPAD, 64), jnp.float32)
    eps_p = jnp.concatenate([eps, jnp.zeros((NPAD - N, 32), jnp.float32)])

    mp_e_n = _make_mp(N, True)
    mp_c = _make_mp(NPAD, False)
    mp_e64 = _make_mp(NPAD, True, 64, False)

    p0, p1 = mp_e_n(x, x, src2d, dst2d, zrows)
    ga, gb = _stage_a(p0, p1, w1cat, w2bd)

    s2a, s2b = mp_c(ga, gb, src2d, dst2d, zrows)
    p = _stage_b(s2a, s2b, wcat[:128], wcat[128:])

    q0, q1 = mp_e64(p, p, src2d, dst2d, zrows64)
    z, m_q_z, std_q_z = _stage_c(q0, q1, eps_p)
    return (z[:N], m_q_z[:N], std_q_z[:N])
